# baseline (speedup 1.0000x reference)
import functools

import jax
import jax.numpy as jnp
from jax import lax
from jax.experimental import pallas as pl
from jax.experimental.pallas import tpu as pltpu
from jax.experimental.pallas import tpu_sc as plsc

_N_PATTERNS = 11
_EMBED = 64
_BATCH = 16384

_info = plsc.get_sparse_core_info()
_NC, _NS, _L = _info.num_cores, _info.num_subcores, _info.num_lanes
_NW = _NC * _NS
_BPW = _BATCH // _NW

_mesh = plsc.VectorSubcoreMesh(core_axis_name="c", subcore_axis_name="s")


@functools.partial(
    pl.kernel,
    mesh=_mesh,
    compiler_params=pltpu.CompilerParams(
        use_tc_tiling_on_sc=True,
        needs_layout_passes=False,
    ),
    out_type=(
        jax.ShapeDtypeStruct((_EMBED, _BATCH), jnp.float32),
        jax.ShapeDtypeStruct((_BATCH,), jnp.float32),
    ),
    scratch_types=[
        pltpu.VMEM((_BPW,), jnp.int32),
        pltpu.VMEM((_N_PATTERNS, _EMBED), jnp.float32),
        pltpu.VMEM((_N_PATTERNS,), jnp.float32),
        pltpu.VMEM((_EMBED, _BPW), jnp.float32),
        pltpu.VMEM((_BPW,), jnp.float32),
        pltpu.SemaphoreType.DMA,
    ],
)
def _sc_lookup(ids_hbm, table_hbm, w_hbm, out_hbm, wout_hbm,
               idx_v, tab_v, wtab_v, rows_v, wout_v, sem):
    wid = lax.axis_index("s") * _NC + lax.axis_index("c")
    base = wid * _BPW
    pltpu.sync_copy(ids_hbm.at[pl.ds(base, _BPW)], idx_v)
    pltpu.sync_copy(table_hbm, tab_v)
    pltpu.sync_copy(w_hbm, wtab_v)

    lane = jnp.arange(_L, dtype=jnp.int32)
    _CQ = 4
    _NQ = _EMBED // _CQ
    _HALF = _BPW // 2

    def _expand(t):
        g = t // _NQ
        q = t % _NQ
        id_vec = idx_v[pl.ds(g * _L, _L)]
        rvec = lane + (g * _L)
        cbase = q * _CQ
        for c in range(_CQ):
            cvec = (lane + c + cbase) & (_EMBED - 1)
            val = plsc.load_gather(tab_v, [id_vec, cvec])
            plsc.store_scatter(rows_v, [cvec, rvec], val)

    _half_iters = (_BPW // _L) * _NQ // 2
    plsc.parallel_loop(0, _half_iters, unroll=1)(_expand)
    cp1 = pltpu.async_copy(
        rows_v.at[:, pl.ds(0, _HALF)], out_hbm.at[:, pl.ds(base, _HALF)], sem
    )
    plsc.parallel_loop(_half_iters, 2 * _half_iters, unroll=1)(_expand)

    @plsc.parallel_loop(0, _BPW // _L, unroll=1)
    def _wgroup(g):
        id_vec = idx_v[pl.ds(g * _L, _L)]
        wout_v[pl.ds(g * _L, _L)] = plsc.load_gather(wtab_v, [id_vec])

    pltpu.sync_copy(wout_v, wout_hbm.at[pl.ds(base, _BPW)])
    pltpu.sync_copy(
        rows_v.at[:, pl.ds(_HALF, _HALF)],
        out_hbm.at[:, pl.ds(base + _HALF, _HALF)],
    )
    cp1.wait()


def kernel(pattern_ids, pattern_embeddings, pattern_weights):
    ids = pattern_ids.astype(jnp.int32)
    emb_t, w = _sc_lookup(ids, pattern_embeddings, pattern_weights)
    return emb_t.T, w


# speedup vs baseline: 1.0520x; 1.0520x over previous
import functools

import jax
import jax.numpy as jnp
from jax import lax
from jax.experimental import pallas as pl
from jax.experimental.pallas import tpu as pltpu
from jax.experimental.pallas import tpu_sc as plsc

_N_PATTERNS = 11
_EMBED = 64
_BATCH = 16384

_info = plsc.get_sparse_core_info()
_NC, _NS, _L = _info.num_cores, _info.num_subcores, _info.num_lanes
_NW = _NC * _NS
_BPW = _BATCH // _NW

_mesh = plsc.VectorSubcoreMesh(core_axis_name="c", subcore_axis_name="s")


@functools.partial(
    pl.kernel,
    mesh=_mesh,
    compiler_params=pltpu.CompilerParams(
        use_tc_tiling_on_sc=True,
        needs_layout_passes=False,
    ),
    out_type=(
        jax.ShapeDtypeStruct((_EMBED, _BATCH), jnp.float32),
        jax.ShapeDtypeStruct((_BATCH,), jnp.float32),
    ),
    scratch_types=[
        pltpu.VMEM((_BPW,), jnp.int32),
        pltpu.VMEM((_N_PATTERNS, _EMBED), jnp.float32),
        pltpu.VMEM((_N_PATTERNS,), jnp.float32),
        pltpu.VMEM((_EMBED, _BPW), jnp.float32),
        pltpu.VMEM((_BPW,), jnp.float32),
    ],
)
def _sc_lookup(ids_hbm, table_hbm, w_hbm, out_hbm, wout_hbm,
               idx_v, tab_v, wtab_v, rows_v, wout_v):
    wid = lax.axis_index("s") * _NC + lax.axis_index("c")
    base = wid * _BPW
    pltpu.sync_copy(ids_hbm.at[pl.ds(base, _BPW)], idx_v)
    pltpu.sync_copy(table_hbm, tab_v)
    pltpu.sync_copy(w_hbm, wtab_v)

    lane = jnp.arange(_L, dtype=jnp.int32)

    _CQ = 8

    @plsc.parallel_loop(0, (_BPW // _L) * (_EMBED // _CQ), unroll=1)
    def _group(t):
        g = t // (_EMBED // _CQ)
        q = t % (_EMBED // _CQ)
        id_vec = idx_v[pl.ds(g * _L, _L)]
        rvec = lane + (g * _L)
        cbase = q * _CQ

        @pl.when(q == 0)
        def _weights():
            wout_v[pl.ds(g * _L, _L)] = plsc.load_gather(wtab_v, [id_vec])

        for c in range(_CQ):
            cvec = (lane + c + cbase) & (_EMBED - 1)
            val = plsc.load_gather(tab_v, [id_vec, cvec])
            plsc.store_scatter(rows_v, [cvec, rvec], val)

    pltpu.sync_copy(wout_v, wout_hbm.at[pl.ds(base, _BPW)])
    pltpu.sync_copy(rows_v, out_hbm.at[:, pl.ds(base, _BPW)])


def kernel(pattern_ids, pattern_embeddings, pattern_weights):
    ids = pattern_ids.astype(jnp.int32)
    emb_t, w = _sc_lookup(ids, pattern_embeddings, pattern_weights)
    return emb_t.T, w


# speedup vs baseline: 1.0886x; 1.0348x over previous
import functools

import jax
import jax.numpy as jnp
from jax import lax
from jax.experimental import pallas as pl
from jax.experimental.pallas import tpu as pltpu
from jax.experimental.pallas import tpu_sc as plsc

_N_PATTERNS = 11
_EMBED = 64
_BATCH = 16384

_info = plsc.get_sparse_core_info()
_NC, _NS, _L = _info.num_cores, _info.num_subcores, _info.num_lanes
_NW = _NC * _NS
_BPW = _BATCH // _NW

_mesh = plsc.VectorSubcoreMesh(core_axis_name="c", subcore_axis_name="s")


@functools.partial(
    pl.kernel,
    mesh=_mesh,
    compiler_params=pltpu.CompilerParams(
        use_tc_tiling_on_sc=True,
        needs_layout_passes=False,
    ),
    out_type=(
        jax.ShapeDtypeStruct((_EMBED, _BATCH), jnp.float32),
        jax.ShapeDtypeStruct((_BATCH,), jnp.float32),
    ),
    scratch_types=[
        pltpu.VMEM((_BPW,), jnp.int32),
        pltpu.VMEM((_N_PATTERNS, _EMBED), jnp.float32),
        pltpu.VMEM((_N_PATTERNS,), jnp.float32),
        pltpu.VMEM((_EMBED, _BPW), jnp.float32),
        pltpu.VMEM((_BPW,), jnp.float32),
        pltpu.SemaphoreType.DMA,
    ],
)
def _sc_lookup(ids_hbm, table_hbm, w_hbm, out_hbm, wout_hbm,
               idx_v, tab_v, wtab_v, rows_v, wout_v, sem):
    wid = lax.axis_index("s") * _NC + lax.axis_index("c")
    base = wid * _BPW
    cp_ids = pltpu.async_copy(ids_hbm.at[pl.ds(base, _BPW)], idx_v, sem)
    cp_tab = pltpu.async_copy(table_hbm, tab_v, sem)
    cp_w = pltpu.async_copy(w_hbm, wtab_v, sem)
    cp_ids.wait()
    cp_tab.wait()
    cp_w.wait()

    lane = jnp.arange(_L, dtype=jnp.int32)

    _CQ = 8

    @plsc.parallel_loop(0, (_BPW // _L) * (_EMBED // _CQ), unroll=1)
    def _group(t):
        g = t // (_EMBED // _CQ)
        q = t % (_EMBED // _CQ)
        id_vec = idx_v[pl.ds(g * _L, _L)]
        rvec = lane + (g * _L)
        cbase = q * _CQ

        @pl.when(q == 0)
        def _weights():
            wout_v[pl.ds(g * _L, _L)] = plsc.load_gather(wtab_v, [id_vec])

        for c in range(_CQ):
            cvec = (lane + c + cbase) & (_EMBED - 1)
            val = plsc.load_gather(tab_v, [id_vec, cvec])
            plsc.store_scatter(rows_v, [cvec, rvec], val)

    pltpu.sync_copy(wout_v, wout_hbm.at[pl.ds(base, _BPW)])
    pltpu.sync_copy(rows_v, out_hbm.at[:, pl.ds(base, _BPW)])


def kernel(pattern_ids, pattern_embeddings, pattern_weights):
    ids = pattern_ids.astype(jnp.int32)
    emb_t, w = _sc_lookup(ids, pattern_embeddings, pattern_weights)
    return emb_t.T, w


# speedup vs baseline: 1.0893x; 1.0007x over previous
import functools

import jax
import jax.numpy as jnp
from jax import lax
from jax.experimental import pallas as pl
from jax.experimental.pallas import tpu as pltpu
from jax.experimental.pallas import tpu_sc as plsc

_N_PATTERNS = 11
_EMBED = 64
_BATCH = 16384

_info = plsc.get_sparse_core_info()
_NC, _NS, _L = _info.num_cores, _info.num_subcores, _info.num_lanes
_NW = _NC * _NS
_BPW = _BATCH // _NW

_mesh = plsc.VectorSubcoreMesh(core_axis_name="c", subcore_axis_name="s")


@functools.partial(
    pl.kernel,
    mesh=_mesh,
    compiler_params=pltpu.CompilerParams(
        use_tc_tiling_on_sc=True,
        needs_layout_passes=False,
    ),
    out_type=(
        jax.ShapeDtypeStruct((_EMBED, _BATCH), jnp.float32),
        jax.ShapeDtypeStruct((_BATCH,), jnp.float32),
    ),
    scratch_types=[
        pltpu.VMEM((_BPW,), jnp.int32),
        pltpu.VMEM((_N_PATTERNS, _EMBED), jnp.float32),
        pltpu.VMEM((_N_PATTERNS,), jnp.float32),
        pltpu.VMEM((_EMBED, _BPW), jnp.float32),
        pltpu.VMEM((_BPW,), jnp.float32),
        pltpu.SemaphoreType.DMA,
    ],
)
def _sc_lookup(ids_hbm, table_hbm, w_hbm, out_hbm, wout_hbm,
               idx_v, tab_v, wtab_v, rows_v, wout_v, sem):
    wid = lax.axis_index("s") * _NC + lax.axis_index("c")
    base = wid * _BPW
    cp_ids = pltpu.async_copy(ids_hbm.at[pl.ds(base, _BPW)], idx_v, sem)
    cp_tab = pltpu.async_copy(table_hbm, tab_v, sem)
    cp_w = pltpu.async_copy(w_hbm, wtab_v, sem)
    cp_ids.wait()
    cp_tab.wait()
    cp_w.wait()

    lane = jnp.arange(_L, dtype=jnp.int32)

    _CQ = 8

    @plsc.parallel_loop(0, (_BPW // _L) * (_EMBED // _CQ), unroll=1)
    def _group(t):
        g = t // (_EMBED // _CQ)
        q = t % (_EMBED // _CQ)
        id_vec = idx_v[pl.ds(g * _L, _L)]
        rvec = lane + (g * _L)
        cbase = q * _CQ

        @pl.when(q == 0)
        def _weights():
            wout_v[pl.ds(g * _L, _L)] = plsc.load_gather(wtab_v, [id_vec])

        for c in range(_CQ):
            cvec = (lane + c + cbase) & (_EMBED - 1)
            val = plsc.load_gather(tab_v, [id_vec, cvec])
            plsc.store_scatter(rows_v, [cvec, rvec], val)

    cp_rows = pltpu.async_copy(rows_v, out_hbm.at[:, pl.ds(base, _BPW)], sem)
    cp_wout = pltpu.async_copy(wout_v, wout_hbm.at[pl.ds(base, _BPW)], sem)
    cp_rows.wait()
    cp_wout.wait()


def kernel(pattern_ids, pattern_embeddings, pattern_weights):
    ids = pattern_ids.astype(jnp.int32)
    emb_t, w = _sc_lookup(ids, pattern_embeddings, pattern_weights)
    return emb_t.T, w
